# baseline (speedup 1.0000x reference)
import functools

import jax
import jax.numpy as jnp
from jax import lax
from jax.experimental import pallas as pl
from jax.experimental.pallas import tpu as pltpu
from jax.experimental.pallas import tpu_sc as plsc

B = 16384
NC, NS = 2, 16
NW = NC * NS
BPW = B // NW
L = 16
NF_G = 32
NF_M = 16
N_ROWS = 1000000
S = 1000448


N_MAIN = 999936
N_TAIL = N_ROWS - N_MAIN
T_MAX = N_MAIN // 128 - 1
RING = 8
RING_M = 2


def _iota16():
    return lax.iota(jnp.int32, L)


def _scalar_at(vec, b):
    return jnp.sum(jnp.where(_iota16() == b, vec, 0))


def _sc_gather_body(uids, iids, gut2, git2, mut2, mit2,
                    gu_o, gi_o, mu_o, mi_o,
                    uix, iix, gu_b, gi_b, mu_b, mi_b,
                    guv, giv, muv, miv, sems, msems, osem):
    wid = lax.axis_index("s") * NC + lax.axis_index("c")
    base = wid * BPW
    pltpu.sync_copy(uids.at[pl.ds(base, BPW)], uix.at[pl.ds(0, BPW)])
    pltpu.sync_copy(iids.at[pl.ds(base, BPW)], iix.at[pl.ds(0, BPW)])

    def fire_g(u, i, b):
        tu = jnp.minimum(lax.shift_right_logical(u, 7), T_MAX) * 128
        ti = jnp.minimum(lax.shift_right_logical(i, 7), T_MAX) * 128
        pltpu.async_copy(gut2.at[:, pl.ds(tu, 128)], gu_b.at[b], sems[b])
        pltpu.async_copy(git2.at[:, pl.ds(ti, 128)], gi_b.at[b], sems[b])

    def fire_m(u, i, m):
        tu = jnp.minimum(lax.shift_right_logical(u, 7), T_MAX) * 128
        ti = jnp.minimum(lax.shift_right_logical(i, 7), T_MAX) * 128
        pltpu.async_copy(mut2.at[:, pl.ds(tu, 128)], mu_b.at[m], msems[m])
        pltpu.async_copy(mit2.at[:, pl.ds(ti, 128)], mi_b.at[m], msems[m])

    def drain_g(b):
        pltpu.make_async_copy(gut2.at[:, pl.ds(0, 128)], gu_b.at[b], sems[b]).wait()
        pltpu.make_async_copy(git2.at[:, pl.ds(0, 128)], gi_b.at[b], sems[b]).wait()

    def drain_m(m):
        pltpu.make_async_copy(mut2.at[:, pl.ds(0, 128)], mu_b.at[m], msems[m]).wait()
        pltpu.make_async_copy(mit2.at[:, pl.ds(0, 128)], mi_b.at[m], msems[m]).wait()

    def extract(u, i, j, b, m):
        lu = jnp.minimum(u - jnp.minimum(lax.shift_right_logical(u, 7), T_MAX) * 128, 127)
        li = jnp.minimum(i - jnp.minimum(lax.shift_right_logical(i, 7), T_MAX) * 128, 127)
        rows = _iota16()
        jcol = jnp.full((L,), j, jnp.int32)
        lus = jnp.full((L,), lu, jnp.int32)
        lis = jnp.full((L,), li, jnp.int32)
        for h in range(2):
            v = plsc.load_gather(gu_b.at[b], [rows + h * L, lus])
            plsc.store_scatter(guv, [rows + h * L, jcol], v)
            v = plsc.load_gather(gi_b.at[b], [rows + h * L, lis])
            plsc.store_scatter(giv, [rows + h * L, jcol], v)
        v = plsc.load_gather(mu_b.at[m], [rows, lus])
        plsc.store_scatter(muv, [rows, jcol], v)
        v = plsc.load_gather(mi_b.at[m], [rows, lis])
        plsc.store_scatter(miv, [rows, jcol], v)

    uv0 = uix[pl.ds(0, L)]
    iv0 = iix[pl.ds(0, L)]
    for b in range(RING):
        fire_g(_scalar_at(uv0, b), _scalar_at(iv0, b), b)
    for m in range(RING_M):
        fire_m(_scalar_at(uv0, m), _scalar_at(iv0, m), m)

    def body(g, carry):
        uvc = uix[pl.ds(8 * g, L)]
        ivc = iix[pl.ds(8 * g, L)]
        for b in range(8):
            j = 8 * g + b
            m = b % RING_M
            drain_g(b)
            drain_m(m)
            extract(_scalar_at(uvc, b), _scalar_at(ivc, b), j, b, m)

            @pl.when(j + RING < BPW)
            def _refire_g():
                fire_g(_scalar_at(uvc, b + RING), _scalar_at(ivc, b + RING), b)

            @pl.when(j + RING_M < BPW)
            def _refire_m():
                fire_m(_scalar_at(uvc, b + RING_M), _scalar_at(ivc, b + RING_M), m)
        return carry

    lax.fori_loop(0, BPW // 8, body, 0)

    out_copies = [
        pltpu.async_copy(guv, gu_o.at[:, pl.ds(base, BPW)], osem),
        pltpu.async_copy(giv, gi_o.at[:, pl.ds(base, BPW)], osem),
        pltpu.async_copy(muv, mu_o.at[:, pl.ds(base, BPW)], osem),
        pltpu.async_copy(miv, mi_o.at[:, pl.ds(base, BPW)], osem),
    ]
    for cp in out_copies:
        cp.wait()


_sc_gather = functools.partial(
    pl.kernel,
    out_type=(
        jax.ShapeDtypeStruct((NF_G, B), jnp.float32),
        jax.ShapeDtypeStruct((NF_G, B), jnp.float32),
        jax.ShapeDtypeStruct((NF_M, B), jnp.float32),
        jax.ShapeDtypeStruct((NF_M, B), jnp.float32),
    ),
    mesh=plsc.VectorSubcoreMesh(core_axis_name="c", subcore_axis_name="s"),
    scratch_types=[
        pltpu.VMEM((BPW + L,), jnp.int32),
        pltpu.VMEM((BPW + L,), jnp.int32),
        pltpu.VMEM((RING, NF_G, 128), jnp.float32),
        pltpu.VMEM((RING, NF_G, 128), jnp.float32),
        pltpu.VMEM((RING_M, NF_M, 128), jnp.float32),
        pltpu.VMEM((RING_M, NF_M, 128), jnp.float32),
        pltpu.VMEM((NF_G, BPW), jnp.float32),
        pltpu.VMEM((NF_G, BPW), jnp.float32),
        pltpu.VMEM((NF_M, BPW), jnp.float32),
        pltpu.VMEM((NF_M, BPW), jnp.float32),
        [pltpu.SemaphoreType.DMA] * RING,
        [pltpu.SemaphoreType.DMA] * RING_M,
        pltpu.SemaphoreType.DMA,
    ],
    compiler_params=pltpu.CompilerParams(needs_layout_passes=False),
)(_sc_gather_body)


def _dense_body(gu, gi, mu, mi, uq, iq, gut_tl, git_tl, mut_tl, mit_tl,
                w1at, w1bt, b1, w2t, b2, w3t, b3,
                wpg, wph, bp, out):
    blk = uq.shape[1]
    iota = lax.broadcasted_iota(jnp.int32, (N_TAIL, blk), 0)
    oh_u = jnp.where(iota == (uq[...] - N_MAIN), 1.0, 0.0)
    oh_i = jnp.where(iota == (iq[...] - N_MAIN), 1.0, 0.0)
    m_u = uq[...] >= N_MAIN
    m_i = iq[...] >= N_MAIN
    gu = jnp.where(m_u, jnp.dot(gut_tl[...], oh_u, preferred_element_type=jnp.float32), gu[...])
    gi = jnp.where(m_i, jnp.dot(git_tl[...], oh_i, preferred_element_type=jnp.float32), gi[...])
    mu = jnp.where(m_u, jnp.dot(mut_tl[...], oh_u, preferred_element_type=jnp.float32), mu[...])
    mi = jnp.where(m_i, jnp.dot(mit_tl[...], oh_i, preferred_element_type=jnp.float32), mi[...])
    h = jnp.dot(w1at[...], mu[...], preferred_element_type=jnp.float32)
    h += jnp.dot(w1bt[...], mi[...], preferred_element_type=jnp.float32)
    h = jnp.maximum(h + b1[...], 0.0)
    h = jnp.maximum(jnp.dot(w2t[...], h, preferred_element_type=jnp.float32) + b2[...], 0.0)
    h = jnp.maximum(jnp.dot(w3t[...], h, preferred_element_type=jnp.float32) + b3[...], 0.0)
    g = gu[...] * gi[...]
    logit = jnp.dot(wpg[...], g, preferred_element_type=jnp.float32)
    logit += jnp.dot(wph[...], h, preferred_element_type=jnp.float32)
    logit += bp[...]
    out[...] = 1.0 / (1.0 + jnp.exp(-logit))


def kernel(user_ids, item_ids, gmf_user_table, gmf_item_table,
           mlp_user_table, mlp_item_table, W1, b1, W2, b2, W3, b3, Wp, bp):
    gu_t, gi_t, mu_t, mi_t = _sc_gather(
        user_ids.astype(jnp.int32), item_ids.astype(jnp.int32),
        gmf_user_table.T, gmf_item_table.T,
        mlp_user_table.T, mlp_item_table.T,
    )
    w1at = W1[:16, :].T
    w1bt = W1[16:, :].T
    uq = user_ids.astype(jnp.int32).reshape(1, B)
    iq = item_ids.astype(jnp.int32).reshape(1, B)
    gut_tl = gmf_user_table[N_MAIN:, :].T
    git_tl = gmf_item_table[N_MAIN:, :].T
    mut_tl = mlp_user_table[N_MAIN:, :].T
    mit_tl = mlp_item_table[N_MAIN:, :].T
    BLK = 4096
    full = lambda shape: pl.BlockSpec(shape, lambda i: (0, 0))
    col = lambda r: pl.BlockSpec((r, BLK), lambda i: (0, i))
    out_t = pl.pallas_call(
        _dense_body,
        grid=(B // BLK,),
        in_specs=[
            col(NF_G), col(NF_G), col(NF_M), col(NF_M),
            col(1), col(1),
            full((NF_G, N_TAIL)), full((NF_G, N_TAIL)),
            full((NF_M, N_TAIL)), full((NF_M, N_TAIL)),
            full((32, 16)), full((32, 16)), full((32, 1)),
            full((16, 32)), full((16, 1)),
            full((8, 16)), full((8, 1)),
            full((1, 32)), full((1, 8)), full((1, 1)),
        ],
        out_specs=col(1),
        out_shape=jax.ShapeDtypeStruct((1, B), jnp.float32),
    )(gu_t, gi_t, mu_t, mi_t, uq, iq, gut_tl, git_tl, mut_tl, mit_tl,
      w1at, w1bt, b1.reshape(32, 1), W2.T,
      b2.reshape(16, 1), W3.T, b3.reshape(8, 1), Wp[:32, 0].reshape(1, 32),
      Wp[32:, 0].reshape(1, 8), bp.reshape(1, 1))
    return out_t.reshape(B, 1)


# speedup vs baseline: 1.1319x; 1.1319x over previous
import functools

import jax
import jax.numpy as jnp
from jax import lax
from jax.experimental import pallas as pl
from jax.experimental.pallas import tpu as pltpu
from jax.experimental.pallas import tpu_sc as plsc

B = 16384
NC, NS = 2, 16
NW = NC * NS
BPW = B // NW
L = 16
NF_G = 32
NF_M = 16
N_ROWS = 1000000
S = 1000448


N_MAIN = 999936
N_TAIL = N_ROWS - N_MAIN
T_MAX = N_MAIN // 128 - 1
RING = 4
RING_M = 4


def _iota16():
    return lax.iota(jnp.int32, L)


def _scalar_at(vec, b):
    return jnp.sum(jnp.where(_iota16() == b, vec, 0))


def _sc_gather_body(uids, iids, gut2, git2, mut2, mit2,
                    gu_o, gi_o, mu_o, mi_o,
                    uix, iix, gu_b, gi_b, mu_b, mi_b,
                    guv, giv, muv, miv, sems, msems, osem):
    wid = lax.axis_index("s") * NC + lax.axis_index("c")
    base = wid * BPW
    pltpu.sync_copy(uids.at[pl.ds(base, BPW)], uix.at[pl.ds(0, BPW)])
    pltpu.sync_copy(iids.at[pl.ds(base, BPW)], iix.at[pl.ds(0, BPW)])

    def fire_g(u, i, b):
        tu = jnp.minimum(lax.shift_right_logical(u, 7), T_MAX) * 128
        ti = jnp.minimum(lax.shift_right_logical(i, 7), T_MAX) * 128
        pltpu.async_copy(gut2.at[:, pl.ds(tu, 128)], gu_b.at[b], sems[b])
        pltpu.async_copy(git2.at[:, pl.ds(ti, 128)], gi_b.at[b], sems[b])

    def fire_m(u, i, m):
        tu = jnp.minimum(lax.shift_right_logical(u, 7), T_MAX) * 128
        ti = jnp.minimum(lax.shift_right_logical(i, 7), T_MAX) * 128
        pltpu.async_copy(mut2.at[:, pl.ds(tu, 128)], mu_b.at[m], msems[m])
        pltpu.async_copy(mit2.at[:, pl.ds(ti, 128)], mi_b.at[m], msems[m])

    def drain_g(b):
        pltpu.make_async_copy(gut2.at[:, pl.ds(0, 128)], gu_b.at[b], sems[b]).wait()
        pltpu.make_async_copy(git2.at[:, pl.ds(0, 128)], gi_b.at[b], sems[b]).wait()

    def drain_m(m):
        pltpu.make_async_copy(mut2.at[:, pl.ds(0, 128)], mu_b.at[m], msems[m]).wait()
        pltpu.make_async_copy(mit2.at[:, pl.ds(0, 128)], mi_b.at[m], msems[m]).wait()

    def extract(u, i, j, b, m):
        lu = jnp.minimum(u - jnp.minimum(lax.shift_right_logical(u, 7), T_MAX) * 128, 127)
        li = jnp.minimum(i - jnp.minimum(lax.shift_right_logical(i, 7), T_MAX) * 128, 127)
        rows = _iota16()
        jcol = jnp.full((L,), j, jnp.int32)
        lus = jnp.full((L,), lu, jnp.int32)
        lis = jnp.full((L,), li, jnp.int32)
        for h in range(2):
            v = plsc.load_gather(gu_b.at[b], [rows + h * L, lus])
            plsc.store_scatter(guv, [rows + h * L, jcol], v)
            v = plsc.load_gather(gi_b.at[b], [rows + h * L, lis])
            plsc.store_scatter(giv, [rows + h * L, jcol], v)
        v = plsc.load_gather(mu_b.at[m], [rows, lus])
        plsc.store_scatter(muv, [rows, jcol], v)
        v = plsc.load_gather(mi_b.at[m], [rows, lis])
        plsc.store_scatter(miv, [rows, jcol], v)

    uv0 = uix[pl.ds(0, L)]
    iv0 = iix[pl.ds(0, L)]
    for b in range(RING):
        fire_g(_scalar_at(uv0, b), _scalar_at(iv0, b), b)
    for m in range(RING_M):
        fire_m(_scalar_at(uv0, m), _scalar_at(iv0, m), m)

    def body(g, carry):
        uvc = uix[pl.ds(8 * g, L)]
        ivc = iix[pl.ds(8 * g, L)]
        for b in range(8):
            j = 8 * g + b
            sg = b % RING
            m = b % RING_M
            drain_g(sg)
            drain_m(m)
            extract(_scalar_at(uvc, b), _scalar_at(ivc, b), j, sg, m)

            @pl.when(j + RING < BPW)
            def _refire_g():
                fire_g(_scalar_at(uvc, b + RING), _scalar_at(ivc, b + RING), sg)

            @pl.when(j + RING_M < BPW)
            def _refire_m():
                fire_m(_scalar_at(uvc, b + RING_M), _scalar_at(ivc, b + RING_M), m)
        return carry

    lax.fori_loop(0, BPW // 8, body, 0)

    out_copies = [
        pltpu.async_copy(guv, gu_o.at[:, pl.ds(base, BPW)], osem),
        pltpu.async_copy(giv, gi_o.at[:, pl.ds(base, BPW)], osem),
        pltpu.async_copy(muv, mu_o.at[:, pl.ds(base, BPW)], osem),
        pltpu.async_copy(miv, mi_o.at[:, pl.ds(base, BPW)], osem),
    ]
    for cp in out_copies:
        cp.wait()


_sc_gather = functools.partial(
    pl.kernel,
    out_type=(
        jax.ShapeDtypeStruct((NF_G, B), jnp.float32),
        jax.ShapeDtypeStruct((NF_G, B), jnp.float32),
        jax.ShapeDtypeStruct((NF_M, B), jnp.float32),
        jax.ShapeDtypeStruct((NF_M, B), jnp.float32),
    ),
    mesh=plsc.VectorSubcoreMesh(core_axis_name="c", subcore_axis_name="s"),
    scratch_types=[
        pltpu.VMEM((BPW + L,), jnp.int32),
        pltpu.VMEM((BPW + L,), jnp.int32),
        pltpu.VMEM((RING, NF_G, 128), jnp.float32),
        pltpu.VMEM((RING, NF_G, 128), jnp.float32),
        pltpu.VMEM((RING_M, NF_M, 128), jnp.float32),
        pltpu.VMEM((RING_M, NF_M, 128), jnp.float32),
        pltpu.VMEM((NF_G, BPW), jnp.float32),
        pltpu.VMEM((NF_G, BPW), jnp.float32),
        pltpu.VMEM((NF_M, BPW), jnp.float32),
        pltpu.VMEM((NF_M, BPW), jnp.float32),
        [pltpu.SemaphoreType.DMA] * RING,
        [pltpu.SemaphoreType.DMA] * RING_M,
        pltpu.SemaphoreType.DMA,
    ],
    compiler_params=pltpu.CompilerParams(needs_layout_passes=False),
)(_sc_gather_body)


def _dense_body(gu, gi, mu, mi, uq, iq, gut_tl, git_tl, mut_tl, mit_tl,
                w1at, w1bt, b1, w2t, b2, w3t, b3,
                wpg, wph, bp, out):
    blk = uq.shape[1]
    iota = lax.broadcasted_iota(jnp.int32, (N_TAIL, blk), 0)
    oh_u = jnp.where(iota == (uq[...] - N_MAIN), 1.0, 0.0)
    oh_i = jnp.where(iota == (iq[...] - N_MAIN), 1.0, 0.0)
    m_u = uq[...] >= N_MAIN
    m_i = iq[...] >= N_MAIN
    gu = jnp.where(m_u, jnp.dot(gut_tl[...], oh_u, preferred_element_type=jnp.float32), gu[...])
    gi = jnp.where(m_i, jnp.dot(git_tl[...], oh_i, preferred_element_type=jnp.float32), gi[...])
    mu = jnp.where(m_u, jnp.dot(mut_tl[...], oh_u, preferred_element_type=jnp.float32), mu[...])
    mi = jnp.where(m_i, jnp.dot(mit_tl[...], oh_i, preferred_element_type=jnp.float32), mi[...])
    h = jnp.dot(w1at[...], mu[...], preferred_element_type=jnp.float32)
    h += jnp.dot(w1bt[...], mi[...], preferred_element_type=jnp.float32)
    h = jnp.maximum(h + b1[...], 0.0)
    h = jnp.maximum(jnp.dot(w2t[...], h, preferred_element_type=jnp.float32) + b2[...], 0.0)
    h = jnp.maximum(jnp.dot(w3t[...], h, preferred_element_type=jnp.float32) + b3[...], 0.0)
    g = gu[...] * gi[...]
    logit = jnp.dot(wpg[...], g, preferred_element_type=jnp.float32)
    logit += jnp.dot(wph[...], h, preferred_element_type=jnp.float32)
    logit += bp[...]
    out[...] = 1.0 / (1.0 + jnp.exp(-logit))


def kernel(user_ids, item_ids, gmf_user_table, gmf_item_table,
           mlp_user_table, mlp_item_table, W1, b1, W2, b2, W3, b3, Wp, bp):
    gu_t, gi_t, mu_t, mi_t = _sc_gather(
        user_ids.astype(jnp.int32), item_ids.astype(jnp.int32),
        gmf_user_table.T, gmf_item_table.T,
        mlp_user_table.T, mlp_item_table.T,
    )
    w1at = W1[:16, :].T
    w1bt = W1[16:, :].T
    uq = user_ids.astype(jnp.int32).reshape(1, B)
    iq = item_ids.astype(jnp.int32).reshape(1, B)
    gut_tl = gmf_user_table[N_MAIN:, :].T
    git_tl = gmf_item_table[N_MAIN:, :].T
    mut_tl = mlp_user_table[N_MAIN:, :].T
    mit_tl = mlp_item_table[N_MAIN:, :].T
    BLK = 4096
    full = lambda shape: pl.BlockSpec(shape, lambda i: (0, 0))
    col = lambda r: pl.BlockSpec((r, BLK), lambda i: (0, i))
    out_t = pl.pallas_call(
        _dense_body,
        grid=(B // BLK,),
        in_specs=[
            col(NF_G), col(NF_G), col(NF_M), col(NF_M),
            col(1), col(1),
            full((NF_G, N_TAIL)), full((NF_G, N_TAIL)),
            full((NF_M, N_TAIL)), full((NF_M, N_TAIL)),
            full((32, 16)), full((32, 16)), full((32, 1)),
            full((16, 32)), full((16, 1)),
            full((8, 16)), full((8, 1)),
            full((1, 32)), full((1, 8)), full((1, 1)),
        ],
        out_specs=col(1),
        out_shape=jax.ShapeDtypeStruct((1, B), jnp.float32),
    )(gu_t, gi_t, mu_t, mi_t, uq, iq, gut_tl, git_tl, mut_tl, mit_tl,
      w1at, w1bt, b1.reshape(32, 1), W2.T,
      b2.reshape(16, 1), W3.T, b3.reshape(8, 1), Wp[:32, 0].reshape(1, 32),
      Wp[32:, 0].reshape(1, 8), bp.reshape(1, 1))
    return out_t.reshape(B, 1)


# speedup vs baseline: 1.1333x; 1.0012x over previous
import functools

import jax
import jax.numpy as jnp
from jax import lax
from jax.experimental import pallas as pl
from jax.experimental.pallas import tpu as pltpu
from jax.experimental.pallas import tpu_sc as plsc

B = 16384
NC, NS = 2, 16
NW = NC * NS
BPW = B // NW
L = 16
NF_G = 32
NF_M = 16
N_ROWS = 1000000
N_MAIN = 999936
N_TAIL = N_ROWS - N_MAIN
T_MAX = N_MAIN // 128 - 1
RING = 4
RING_M = 4


def _iota16():
    return lax.iota(jnp.int32, L)


def _scalar_at(vec, b):
    return jnp.sum(jnp.where(_iota16() == b, vec, 0))


def _sc_gather_body(uids, iids, gut2, git2, mut2, mit2,
                    gu_o, gi_o, mu_o, mi_o,
                    uix, iix, gu_b, gi_b, mu_b, mi_b,
                    guv, giv, muv, miv, sems, msems, osem):
    wid = lax.axis_index("s") * NC + lax.axis_index("c")
    base = wid * BPW
    pltpu.sync_copy(uids.at[pl.ds(base, BPW)], uix.at[pl.ds(0, BPW)])
    pltpu.sync_copy(iids.at[pl.ds(base, BPW)], iix.at[pl.ds(0, BPW)])

    def fire_g(u, i, b):
        tu = jnp.minimum(lax.shift_right_logical(u, 7), T_MAX) * 128
        ti = jnp.minimum(lax.shift_right_logical(i, 7), T_MAX) * 128
        pltpu.async_copy(gut2.at[:, pl.ds(tu, 128)], gu_b.at[b], sems[b])
        pltpu.async_copy(git2.at[:, pl.ds(ti, 128)], gi_b.at[b], sems[b])

    def fire_m(u, i, m):
        tu = jnp.minimum(lax.shift_right_logical(u, 7), T_MAX) * 128
        ti = jnp.minimum(lax.shift_right_logical(i, 7), T_MAX) * 128
        pltpu.async_copy(mut2.at[:, pl.ds(tu, 128)], mu_b.at[m], msems[m])
        pltpu.async_copy(mit2.at[:, pl.ds(ti, 128)], mi_b.at[m], msems[m])

    def drain_g(b):
        pltpu.make_async_copy(gut2.at[:, pl.ds(0, 128)], gu_b.at[b], sems[b]).wait()
        pltpu.make_async_copy(git2.at[:, pl.ds(0, 128)], gi_b.at[b], sems[b]).wait()

    def drain_m(m):
        pltpu.make_async_copy(mut2.at[:, pl.ds(0, 128)], mu_b.at[m], msems[m]).wait()
        pltpu.make_async_copy(mit2.at[:, pl.ds(0, 128)], mi_b.at[m], msems[m]).wait()

    def extract(u, i, j, b, m):
        lu = jnp.minimum(u - jnp.minimum(lax.shift_right_logical(u, 7), T_MAX) * 128, 127)
        li = jnp.minimum(i - jnp.minimum(lax.shift_right_logical(i, 7), T_MAX) * 128, 127)
        rows = _iota16()
        jcol = jnp.full((L,), j, jnp.int32)
        lus = jnp.full((L,), lu, jnp.int32)
        lis = jnp.full((L,), li, jnp.int32)
        for h in range(2):
            v = plsc.load_gather(gu_b.at[b], [rows + h * L, lus])
            plsc.store_scatter(guv, [rows + h * L, jcol], v)
            v = plsc.load_gather(gi_b.at[b], [rows + h * L, lis])
            plsc.store_scatter(giv, [rows + h * L, jcol], v)
        v = plsc.load_gather(mu_b.at[m], [rows, lus])
        plsc.store_scatter(muv, [rows, jcol], v)
        v = plsc.load_gather(mi_b.at[m], [rows, lis])
        plsc.store_scatter(miv, [rows, jcol], v)

    uv0 = uix[pl.ds(0, L)]
    iv0 = iix[pl.ds(0, L)]
    for b in range(RING):
        fire_g(_scalar_at(uv0, b), _scalar_at(iv0, b), b)
    for m in range(RING_M):
        fire_m(_scalar_at(uv0, m), _scalar_at(iv0, m), m)

    def body(g, carry):
        uvc = uix[pl.ds(8 * g, L)]
        ivc = iix[pl.ds(8 * g, L)]
        for b in range(8):
            j = 8 * g + b
            sg = b % RING
            m = b % RING_M
            drain_g(sg)
            drain_m(m)
            extract(_scalar_at(uvc, b), _scalar_at(ivc, b), j, sg, m)

            @pl.when(j + RING < BPW)
            def _refire_g():
                fire_g(_scalar_at(uvc, b + RING), _scalar_at(ivc, b + RING), sg)

            @pl.when(j + RING_M < BPW)
            def _refire_m():
                fire_m(_scalar_at(uvc, b + RING_M), _scalar_at(ivc, b + RING_M), m)
        return carry

    lax.fori_loop(0, BPW // 8, body, 0)

    out_copies = [
        pltpu.async_copy(guv, gu_o.at[:, pl.ds(base, BPW)], osem),
        pltpu.async_copy(giv, gi_o.at[:, pl.ds(base, BPW)], osem),
        pltpu.async_copy(muv, mu_o.at[:, pl.ds(base, BPW)], osem),
        pltpu.async_copy(miv, mi_o.at[:, pl.ds(base, BPW)], osem),
    ]
    for cp in out_copies:
        cp.wait()


_sc_gather = functools.partial(
    pl.kernel,
    out_type=(
        jax.ShapeDtypeStruct((NF_G, B), jnp.float32),
        jax.ShapeDtypeStruct((NF_G, B), jnp.float32),
        jax.ShapeDtypeStruct((NF_M, B), jnp.float32),
        jax.ShapeDtypeStruct((NF_M, B), jnp.float32),
    ),
    mesh=plsc.VectorSubcoreMesh(core_axis_name="c", subcore_axis_name="s"),
    scratch_types=[
        pltpu.VMEM((BPW + L,), jnp.int32),
        pltpu.VMEM((BPW + L,), jnp.int32),
        pltpu.VMEM((RING, NF_G, 128), jnp.float32),
        pltpu.VMEM((RING, NF_G, 128), jnp.float32),
        pltpu.VMEM((RING_M, NF_M, 128), jnp.float32),
        pltpu.VMEM((RING_M, NF_M, 128), jnp.float32),
        pltpu.VMEM((NF_G, BPW), jnp.float32),
        pltpu.VMEM((NF_G, BPW), jnp.float32),
        pltpu.VMEM((NF_M, BPW), jnp.float32),
        pltpu.VMEM((NF_M, BPW), jnp.float32),
        [pltpu.SemaphoreType.DMA] * RING,
        [pltpu.SemaphoreType.DMA] * RING_M,
        pltpu.SemaphoreType.DMA,
    ],
    compiler_params=pltpu.CompilerParams(needs_layout_passes=False),
)(_sc_gather_body)


def _dense_body(gu, gi, mu, mi, uq, iq, gut_tl, git_tl, mut_tl, mit_tl,
                w1at, w1bt, b1, w2t, b2, w3t, b3,
                wpg, wph, bp, out):
    blk = uq.shape[1]
    iota = lax.broadcasted_iota(jnp.int32, (N_TAIL, blk), 0)
    oh_u = jnp.where(iota == (uq[...] - N_MAIN), 1.0, 0.0)
    oh_i = jnp.where(iota == (iq[...] - N_MAIN), 1.0, 0.0)
    m_u = uq[...] >= N_MAIN
    m_i = iq[...] >= N_MAIN
    gu = jnp.where(m_u, jnp.dot(gut_tl[...], oh_u, preferred_element_type=jnp.float32), gu[...])
    gi = jnp.where(m_i, jnp.dot(git_tl[...], oh_i, preferred_element_type=jnp.float32), gi[...])
    mu = jnp.where(m_u, jnp.dot(mut_tl[...], oh_u, preferred_element_type=jnp.float32), mu[...])
    mi = jnp.where(m_i, jnp.dot(mit_tl[...], oh_i, preferred_element_type=jnp.float32), mi[...])
    h = jnp.dot(w1at[...], mu[...], preferred_element_type=jnp.float32)
    h += jnp.dot(w1bt[...], mi[...], preferred_element_type=jnp.float32)
    h = jnp.maximum(h + b1[...], 0.0)
    h = jnp.maximum(jnp.dot(w2t[...], h, preferred_element_type=jnp.float32) + b2[...], 0.0)
    h = jnp.maximum(jnp.dot(w3t[...], h, preferred_element_type=jnp.float32) + b3[...], 0.0)
    g = gu[...] * gi[...]
    logit = jnp.dot(wpg[...], g, preferred_element_type=jnp.float32)
    logit += jnp.dot(wph[...], h, preferred_element_type=jnp.float32)
    logit += bp[...]
    out[...] = 1.0 / (1.0 + jnp.exp(-logit))


def kernel(user_ids, item_ids, gmf_user_table, gmf_item_table,
           mlp_user_table, mlp_item_table, W1, b1, W2, b2, W3, b3, Wp, bp):
    gu_t, gi_t, mu_t, mi_t = _sc_gather(
        user_ids.astype(jnp.int32), item_ids.astype(jnp.int32),
        gmf_user_table.T, gmf_item_table.T,
        mlp_user_table.T, mlp_item_table.T,
    )
    w1at = W1[:16, :].T
    w1bt = W1[16:, :].T
    uq = user_ids.astype(jnp.int32).reshape(1, B)
    iq = item_ids.astype(jnp.int32).reshape(1, B)
    gut_tl = gmf_user_table[N_MAIN:, :].T
    git_tl = gmf_item_table[N_MAIN:, :].T
    mut_tl = mlp_user_table[N_MAIN:, :].T
    mit_tl = mlp_item_table[N_MAIN:, :].T
    BLK = 4096
    full = lambda shape: pl.BlockSpec(shape, lambda i: (0, 0))
    col = lambda r: pl.BlockSpec((r, BLK), lambda i: (0, i))
    out_t = pl.pallas_call(
        _dense_body,
        grid=(B // BLK,),
        in_specs=[
            col(NF_G), col(NF_G), col(NF_M), col(NF_M),
            col(1), col(1),
            full((NF_G, N_TAIL)), full((NF_G, N_TAIL)),
            full((NF_M, N_TAIL)), full((NF_M, N_TAIL)),
            full((32, 16)), full((32, 16)), full((32, 1)),
            full((16, 32)), full((16, 1)),
            full((8, 16)), full((8, 1)),
            full((1, 32)), full((1, 8)), full((1, 1)),
        ],
        out_specs=col(1),
        out_shape=jax.ShapeDtypeStruct((1, B), jnp.float32),
    )(gu_t, gi_t, mu_t, mi_t, uq, iq, gut_tl, git_tl, mut_tl, mit_tl,
      w1at, w1bt, b1.reshape(32, 1), W2.T,
      b2.reshape(16, 1), W3.T, b3.reshape(8, 1), Wp[:32, 0].reshape(1, 32),
      Wp[32:, 0].reshape(1, 8), bp.reshape(1, 1))
    return out_t.reshape(B, 1)
